# Initial kernel scaffold; baseline (speedup 1.0000x reference)
import jax
import jax.numpy as jnp
from jax import lax
from jax.experimental import pallas as pl
from jax.experimental.pallas import tpu as pltpu
from jax.experimental.pallas import tpu_sc as plsc

_N = 10000
_E = 320000
_D = 128
_G = 64

_NC = 2
_NS = 16
_EPW = _E // (_NC * _NS)
_CHUNK = 100
_NCHUNK = _EPW // _CHUNK
_RPT = _N // _NS

_BLK = 1000
_R = _N // _BLK


def _sc_agg_body(h_hbm, src_hbm, dst_hbm, zero_hbm, out_hbm,
                 src_v, dst_v, rows_v, acc_sh, sem):
    c = lax.axis_index("c")
    s = lax.axis_index("s")
    pltpu.sync_copy(src_hbm.at[c, s], src_v)
    pltpu.sync_copy(dst_hbm.at[c, s], dst_v)
    r0 = s * _RPT
    pltpu.sync_copy(zero_hbm.at[pl.ds(r0, _RPT)], acc_sh.at[pl.ds(r0, _RPT)])
    plsc.subcore_barrier()

    def body(j, carry):
        pltpu.async_copy(h_hbm.at[src_v.at[j]], rows_v, sem).wait()
        pltpu.sync_copy(rows_v, acc_sh.at[dst_v.at[j]], add=True)
        return carry

    lax.fori_loop(0, _NCHUNK, body, 0)
    plsc.subcore_barrier()
    pltpu.sync_copy(acc_sh.at[pl.ds(r0, _RPT)], out_hbm.at[c, pl.ds(r0, _RPT)])


_sc_agg = pl.kernel(
    _sc_agg_body,
    out_type=jax.ShapeDtypeStruct((_NC, _N, _D), jnp.float32),
    mesh=plsc.VectorSubcoreMesh(core_axis_name="c", subcore_axis_name="s"),
    scratch_types=[
        pltpu.VMEM((_NCHUNK, _CHUNK), jnp.int32),
        pltpu.VMEM((_NCHUNK, _CHUNK), jnp.int32),
        pltpu.VMEM((_CHUNK, _D), jnp.float32),
        pltpu.VMEM_SHARED((_N, _D), jnp.float32),
        pltpu.SemaphoreType.DMA,
    ],
)


def _mlp_body(h_ref, agg_ref, w1_ref, b1_ref, w2_ref, b2_ref, z_ref, st_ref):
    i = pl.program_id(0)
    sm = h_ref[...] + agg_ref[0] + agg_ref[1]
    a = jnp.maximum(
        jnp.dot(sm, w1_ref[...], preferred_element_type=jnp.float32)
        + b1_ref[...], 0.0)
    z = (jnp.dot(a, w2_ref[...], preferred_element_type=jnp.float32)
         + b2_ref[...])
    z_ref[...] = z
    part = jnp.concatenate(
        [jnp.sum(z, axis=0, keepdims=True),
         jnp.sum(z * z, axis=0, keepdims=True),
         jnp.zeros((6, _D), jnp.float32)], axis=0)

    @pl.when(i == 0)
    def _():
        st_ref[...] = part

    @pl.when(i > 0)
    def _():
        st_ref[...] += part


_mlp = pl.pallas_call(
    _mlp_body,
    grid=(_R,),
    in_specs=[
        pl.BlockSpec((_BLK, _D), lambda i: (i, 0)),
        pl.BlockSpec((_NC, _BLK, _D), lambda i: (0, i, 0)),
        pl.BlockSpec((_D, _D), lambda i: (0, 0)),
        pl.BlockSpec((1, _D), lambda i: (0, 0)),
        pl.BlockSpec((_D, _D), lambda i: (0, 0)),
        pl.BlockSpec((1, _D), lambda i: (0, 0)),
    ],
    out_specs=[
        pl.BlockSpec((_BLK, _D), lambda i: (i, 0)),
        pl.BlockSpec((8, _D), lambda i: (0, 0)),
    ],
    out_shape=[
        jax.ShapeDtypeStruct((_N, _D), jnp.float32),
        jax.ShapeDtypeStruct((8, _D), jnp.float32),
    ],
)


def _bn_stats(st_ref):
    m = st_ref[0:1] / _N
    v = st_ref[1:2] / _N - m * m
    inv = lax.rsqrt(v + 1e-5)
    return m, inv


def _bn_body(z_ref, st_ref, g_ref, b_ref, o_ref):
    m, inv = _bn_stats(st_ref)
    o_ref[...] = jnp.maximum(
        (z_ref[...] - m) * inv * g_ref[...] + b_ref[...], 0.0)


_bn = pl.pallas_call(
    _bn_body,
    grid=(_R,),
    in_specs=[
        pl.BlockSpec((_BLK, _D), lambda i: (i, 0)),
        pl.BlockSpec((8, _D), lambda i: (0, 0)),
        pl.BlockSpec((1, _D), lambda i: (0, 0)),
        pl.BlockSpec((1, _D), lambda i: (0, 0)),
    ],
    out_specs=pl.BlockSpec((_BLK, _D), lambda i: (i, 0)),
    out_shape=jax.ShapeDtypeStruct((_N, _D), jnp.float32),
)


def _bnpool_body(z_ref, st_ref, g_ref, b_ref, bat_ref, o_ref, cnt_ref):
    i = pl.program_id(0)
    m, inv = _bn_stats(st_ref)
    zn = jnp.maximum((z_ref[...] - m) * inv * g_ref[...] + b_ref[...], 0.0)
    bvec = bat_ref[0, 0, :]
    gid = lax.broadcasted_iota(jnp.int32, (_BLK, _G), 1)
    mask = (bvec[:, None] == gid).astype(jnp.float32)
    pool = lax.dot_general(mask, zn, (((0,), (0,)), ((), ())),
                           preferred_element_type=jnp.float32)
    cnt = lax.dot_general(mask, jnp.ones((_BLK, _D), jnp.float32),
                          (((0,), (0,)), ((), ())),
                          preferred_element_type=jnp.float32)

    @pl.when(i == 0)
    def _():
        o_ref[...] = pool
        cnt_ref[...] = cnt

    @pl.when(i > 0)
    def _():
        o_ref[...] += pool
        cnt_ref[...] += cnt

    @pl.when(i == _R - 1)
    def _():
        o_ref[...] = o_ref[...] / jnp.maximum(cnt_ref[...], 1.0)


_bnpool = pl.pallas_call(
    _bnpool_body,
    grid=(_R,),
    in_specs=[
        pl.BlockSpec((_BLK, _D), lambda i: (i, 0)),
        pl.BlockSpec((8, _D), lambda i: (0, 0)),
        pl.BlockSpec((1, _D), lambda i: (0, 0)),
        pl.BlockSpec((1, _D), lambda i: (0, 0)),
        pl.BlockSpec((1, 1, _BLK), lambda i: (i, 0, 0)),
    ],
    out_specs=pl.BlockSpec((_G, _D), lambda i: (0, 0)),
    out_shape=jax.ShapeDtypeStruct((_G, _D), jnp.float32),
    scratch_shapes=[pltpu.VMEM((_G, _D), jnp.float32)],
)


def kernel(x, edge_index, batch,
           w1_1, b1_1, w1_2, b1_2,
           w2_1, b2_1, w2_2, b2_2,
           w3_1, b3_1, w3_2, b3_2,
           bn1_g, bn1_b, bn2_g, bn2_b, bn3_g, bn3_b):
    src = edge_index[0].reshape(_NC, _NS, _NCHUNK, _CHUNK)
    dst = edge_index[1].reshape(_NC, _NS, _NCHUNK, _CHUNK)
    zero = jnp.zeros((_N, _D), jnp.float32)
    bat3 = batch.reshape(_R, 1, _BLK)

    layers = [
        (w1_1, b1_1, w1_2, b1_2),
        (w2_1, b2_1, w2_2, b2_2),
        (w3_1, b3_1, w3_2, b3_2),
    ]
    bns = [(bn1_g, bn1_b), (bn2_g, bn2_b)]
    h = x
    for li, (wa, ba, wb, bb) in enumerate(layers):
        agg = _sc_agg(h, src, dst, zero)
        z, st = _mlp(h, agg, wa, ba.reshape(1, _D), wb, bb.reshape(1, _D))
        if li < 2:
            g, b = bns[li]
            h = _bn(z, st, g.reshape(1, _D), b.reshape(1, _D))
    return _bnpool(z, st, bn3_g.reshape(1, _D), bn3_b.reshape(1, _D), bat3)


# speedup vs baseline: 6.9317x; 6.9317x over previous
import functools

import jax
import jax.numpy as jnp
from jax import lax
from jax.experimental import pallas as pl
from jax.experimental.pallas import tpu as pltpu
from jax.experimental.pallas import tpu_sc as plsc

_N = 10000
_E = 320000
_D = 128
_G = 64

_NC = 2
_NS = 16
_EPW = _E // (_NC * _NS)
_CHUNK = 100
_NCHUNK = _EPW // _CHUNK
_NP = 10240
_RPT = _NP // _NS

_BLK = 1000
_R = _N // _BLK


def _sc_agg_body(h_hbm, src_hbm, dst_hbm, zero_hbm, out_hbm,
                 src_v, dst_v, rows_v, acc_sh, sem):
    c = lax.axis_index("c")
    s = lax.axis_index("s")
    pltpu.sync_copy(src_hbm.at[c, s], src_v)
    pltpu.sync_copy(dst_hbm.at[c, s], dst_v)
    r0 = s * _RPT
    pltpu.sync_copy(zero_hbm.at[pl.ds(r0, _RPT)], acc_sh.at[pl.ds(r0, _RPT)])
    plsc.subcore_barrier()

    def body(j, carry):
        pltpu.async_copy(h_hbm.at[src_v.at[j]], rows_v, sem).wait()
        pltpu.sync_copy(rows_v, acc_sh.at[dst_v.at[j]], add=True)
        return carry

    lax.fori_loop(0, _NCHUNK, body, 0)
    plsc.subcore_barrier()
    pltpu.sync_copy(acc_sh.at[pl.ds(r0, _RPT)], out_hbm.at[c, pl.ds(r0, _RPT)])


@functools.lru_cache(maxsize=None)
def _get_sc_agg():
    return pl.kernel(
        _sc_agg_body,
        out_type=jax.ShapeDtypeStruct((_NC, _NP, _D), jnp.float32),
        mesh=plsc.VectorSubcoreMesh(
            core_axis_name="c", subcore_axis_name="s", num_cores=_NC),
        scratch_types=[
            pltpu.VMEM((_NCHUNK, _CHUNK), jnp.int32),
            pltpu.VMEM((_NCHUNK, _CHUNK), jnp.int32),
            pltpu.VMEM((_CHUNK, _D), jnp.float32),
            pltpu.VMEM_SHARED((_NP, _D), jnp.float32),
            pltpu.SemaphoreType.DMA,
        ],
    )


def _mlp_body(h_ref, agg_ref, w1_ref, b1_ref, w2_ref, b2_ref, z_ref, st_ref):
    i = pl.program_id(0)
    sm = h_ref[...] + agg_ref[0] + agg_ref[1]
    a = jnp.maximum(
        jnp.dot(sm, w1_ref[...], preferred_element_type=jnp.float32)
        + b1_ref[...], 0.0)
    z = (jnp.dot(a, w2_ref[...], preferred_element_type=jnp.float32)
         + b2_ref[...])
    z_ref[...] = z
    part = jnp.concatenate(
        [jnp.sum(z, axis=0, keepdims=True),
         jnp.sum(z * z, axis=0, keepdims=True),
         jnp.zeros((6, _D), jnp.float32)], axis=0)

    @pl.when(i == 0)
    def _():
        st_ref[...] = part

    @pl.when(i > 0)
    def _():
        st_ref[...] += part


_mlp = pl.pallas_call(
    _mlp_body,
    grid=(_R,),
    in_specs=[
        pl.BlockSpec((_BLK, _D), lambda i: (i, 0)),
        pl.BlockSpec((_NC, _BLK, _D), lambda i: (0, i, 0)),
        pl.BlockSpec((_D, _D), lambda i: (0, 0)),
        pl.BlockSpec((1, _D), lambda i: (0, 0)),
        pl.BlockSpec((_D, _D), lambda i: (0, 0)),
        pl.BlockSpec((1, _D), lambda i: (0, 0)),
    ],
    out_specs=[
        pl.BlockSpec((_BLK, _D), lambda i: (i, 0)),
        pl.BlockSpec((8, _D), lambda i: (0, 0)),
    ],
    out_shape=[
        jax.ShapeDtypeStruct((_N, _D), jnp.float32),
        jax.ShapeDtypeStruct((8, _D), jnp.float32),
    ],
)


def _bn_stats(st_ref):
    m = st_ref[0:1] / _N
    v = st_ref[1:2] / _N - m * m
    inv = lax.rsqrt(v + 1e-5)
    return m, inv


def _bn_body(z_ref, st_ref, g_ref, b_ref, o_ref):
    m, inv = _bn_stats(st_ref)
    o_ref[...] = jnp.maximum(
        (z_ref[...] - m) * inv * g_ref[...] + b_ref[...], 0.0)


_bn = pl.pallas_call(
    _bn_body,
    grid=(_R,),
    in_specs=[
        pl.BlockSpec((_BLK, _D), lambda i: (i, 0)),
        pl.BlockSpec((8, _D), lambda i: (0, 0)),
        pl.BlockSpec((1, _D), lambda i: (0, 0)),
        pl.BlockSpec((1, _D), lambda i: (0, 0)),
    ],
    out_specs=pl.BlockSpec((_BLK, _D), lambda i: (i, 0)),
    out_shape=jax.ShapeDtypeStruct((_N, _D), jnp.float32),
)


def _bnpool_body(z_ref, st_ref, g_ref, b_ref, bat_ref, o_ref, cnt_ref):
    i = pl.program_id(0)
    m, inv = _bn_stats(st_ref)
    zn = jnp.maximum((z_ref[...] - m) * inv * g_ref[...] + b_ref[...], 0.0)
    bvec = bat_ref[0, 0, :]
    gid = lax.broadcasted_iota(jnp.int32, (_BLK, _G), 1)
    mask = (bvec[:, None] == gid).astype(jnp.float32)
    pool = lax.dot_general(mask, zn, (((0,), (0,)), ((), ())),
                           preferred_element_type=jnp.float32)
    cnt = lax.dot_general(mask, jnp.ones((_BLK, _D), jnp.float32),
                          (((0,), (0,)), ((), ())),
                          preferred_element_type=jnp.float32)

    @pl.when(i == 0)
    def _():
        o_ref[...] = pool
        cnt_ref[...] = cnt

    @pl.when(i > 0)
    def _():
        o_ref[...] += pool
        cnt_ref[...] += cnt

    @pl.when(i == _R - 1)
    def _():
        o_ref[...] = o_ref[...] / jnp.maximum(cnt_ref[...], 1.0)


_bnpool = pl.pallas_call(
    _bnpool_body,
    grid=(_R,),
    in_specs=[
        pl.BlockSpec((_BLK, _D), lambda i: (i, 0)),
        pl.BlockSpec((8, _D), lambda i: (0, 0)),
        pl.BlockSpec((1, _D), lambda i: (0, 0)),
        pl.BlockSpec((1, _D), lambda i: (0, 0)),
        pl.BlockSpec((1, 1, _BLK), lambda i: (i, 0, 0)),
    ],
    out_specs=pl.BlockSpec((_G, _D), lambda i: (0, 0)),
    out_shape=jax.ShapeDtypeStruct((_G, _D), jnp.float32),
    scratch_shapes=[pltpu.VMEM((_G, _D), jnp.float32)],
)


def kernel(x, edge_index, batch,
           w1_1, b1_1, w1_2, b1_2,
           w2_1, b2_1, w2_2, b2_2,
           w3_1, b3_1, w3_2, b3_2,
           bn1_g, bn1_b, bn2_g, bn2_b, bn3_g, bn3_b):
    src = edge_index[0].reshape(_NC, _NS, _NCHUNK, _CHUNK)
    dst = edge_index[1].reshape(_NC, _NS, _NCHUNK, _CHUNK)
    zero = jnp.zeros((_NP, _D), jnp.float32)
    bat3 = batch.reshape(_R, 1, _BLK)

    layers = [
        (w1_1, b1_1, w1_2, b1_2),
        (w2_1, b2_1, w2_2, b2_2),
        (w3_1, b3_1, w3_2, b3_2),
    ]
    bns = [(bn1_g, bn1_b), (bn2_g, bn2_b)]
    h = x
    for li, (wa, ba, wb, bb) in enumerate(layers):
        agg = _get_sc_agg()(h, src, dst, zero)
        z, st = _mlp(h, agg, wa, ba.reshape(1, _D), wb, bb.reshape(1, _D))
        if li < 2:
            g, b = bns[li]
            h = _bn(z, st, g.reshape(1, _D), b.reshape(1, _D))
    return _bnpool(z, st, bn3_g.reshape(1, _D), bn3_b.reshape(1, _D), bat3)


# speedup vs baseline: 10.5302x; 1.5191x over previous
import functools

import jax
import jax.numpy as jnp
from jax import lax
from jax.experimental import pallas as pl
from jax.experimental.pallas import tpu as pltpu
from jax.experimental.pallas import tpu_sc as plsc

_N = 10000
_E = 320000
_D = 128
_G = 64

_NC = 2
_NS = 16
_EPW = _E // (_NC * _NS)
_CHUNK = 100
_NCHUNK = _EPW // _CHUNK
_HCHUNK = _NCHUNK // 2
_NP = 10240
_RPT = _NP // _NS
_HTAIL = _N - (_NS - 1) * _RPT

_BLK = 1000
_R = _N // _BLK


def _sc_agg_body(h_hbm, src_hbm, dst_hbm, zero_hbm, out_hbm,
                 src_v, dst_v, buf0, buf1, acc_sh, sem0, sem1):
    c = lax.axis_index("c")
    s = lax.axis_index("s")
    r0 = s * _RPT

    @pl.when(c == 0)
    def _():
        @pl.when(s < _NS - 1)
        def _():
            pltpu.sync_copy(h_hbm.at[pl.ds(r0, _RPT)],
                            acc_sh.at[pl.ds(r0, _RPT)])

        @pl.when(s == _NS - 1)
        def _():
            pltpu.sync_copy(h_hbm.at[pl.ds(_N - _HTAIL, _HTAIL)],
                            acc_sh.at[pl.ds(_N - _HTAIL, _HTAIL)])
            pltpu.sync_copy(zero_hbm.at[pl.ds(0, _NP - _N)],
                            acc_sh.at[pl.ds(_N, _NP - _N)])

    @pl.when(c == 1)
    def _():
        pltpu.sync_copy(zero_hbm.at[pl.ds(r0, _RPT)],
                        acc_sh.at[pl.ds(r0, _RPT)])

    plsc.subcore_barrier()

    for half in range(2):
        pltpu.sync_copy(src_hbm.at[half, c, s], src_v)
        pltpu.sync_copy(dst_hbm.at[half, c, s], dst_v)
        pltpu.async_copy(h_hbm.at[src_v.at[0]], buf0, sem0)
        pltpu.async_copy(h_hbm.at[src_v.at[1]], buf1, sem1)

        def body(p, carry):
            j = 2 * p
            pltpu.make_async_copy(h_hbm.at[src_v.at[j]], buf0, sem0).wait()
            pltpu.sync_copy(buf0, acc_sh.at[dst_v.at[j]], add=True)
            pltpu.async_copy(h_hbm.at[src_v.at[j + 2]], buf0, sem0)
            pltpu.make_async_copy(h_hbm.at[src_v.at[j + 1]], buf1,
                                  sem1).wait()
            pltpu.sync_copy(buf1, acc_sh.at[dst_v.at[j + 1]], add=True)
            pltpu.async_copy(h_hbm.at[src_v.at[j + 3]], buf1, sem1)
            return carry

        lax.fori_loop(0, _HCHUNK // 2 - 1, body, 0)
        j = _HCHUNK - 2
        pltpu.make_async_copy(h_hbm.at[src_v.at[j]], buf0, sem0).wait()
        pltpu.sync_copy(buf0, acc_sh.at[dst_v.at[j]], add=True)
        pltpu.make_async_copy(h_hbm.at[src_v.at[j + 1]], buf1, sem1).wait()
        pltpu.sync_copy(buf1, acc_sh.at[dst_v.at[j + 1]], add=True)

    plsc.subcore_barrier()
    pltpu.sync_copy(acc_sh.at[pl.ds(r0, _RPT)], out_hbm.at[c, pl.ds(r0, _RPT)])


@functools.lru_cache(maxsize=None)
def _get_sc_agg():
    return pl.kernel(
        _sc_agg_body,
        out_type=jax.ShapeDtypeStruct((_NC, _NP, _D), jnp.float32),
        mesh=plsc.VectorSubcoreMesh(
            core_axis_name="c", subcore_axis_name="s", num_cores=_NC),
        scratch_types=[
            pltpu.VMEM((_HCHUNK, _CHUNK), jnp.int32),
            pltpu.VMEM((_HCHUNK, _CHUNK), jnp.int32),
            pltpu.VMEM((_CHUNK, _D), jnp.float32),
            pltpu.VMEM((_CHUNK, _D), jnp.float32),
            pltpu.VMEM_SHARED((_NP, _D), jnp.float32),
            pltpu.SemaphoreType.DMA,
            pltpu.SemaphoreType.DMA,
        ],
    )


def _mlp_body(agg_ref, w1_ref, b1_ref, w2_ref, b2_ref, z_ref, st_ref):
    i = pl.program_id(0)
    sm = agg_ref[0] + agg_ref[1]
    a = jnp.maximum(
        jnp.dot(sm, w1_ref[...], preferred_element_type=jnp.float32)
        + b1_ref[...], 0.0)
    z = (jnp.dot(a, w2_ref[...], preferred_element_type=jnp.float32)
         + b2_ref[...])
    z_ref[...] = z
    part = jnp.concatenate(
        [jnp.sum(z, axis=0, keepdims=True),
         jnp.sum(z * z, axis=0, keepdims=True),
         jnp.zeros((6, _D), jnp.float32)], axis=0)

    @pl.when(i == 0)
    def _():
        st_ref[...] = part

    @pl.when(i > 0)
    def _():
        st_ref[...] += part


_mlp = pl.pallas_call(
    _mlp_body,
    grid=(_R,),
    in_specs=[
        pl.BlockSpec((_NC, _BLK, _D), lambda i: (0, i, 0)),
        pl.BlockSpec((_D, _D), lambda i: (0, 0)),
        pl.BlockSpec((1, _D), lambda i: (0, 0)),
        pl.BlockSpec((_D, _D), lambda i: (0, 0)),
        pl.BlockSpec((1, _D), lambda i: (0, 0)),
    ],
    out_specs=[
        pl.BlockSpec((_BLK, _D), lambda i: (i, 0)),
        pl.BlockSpec((8, _D), lambda i: (0, 0)),
    ],
    out_shape=[
        jax.ShapeDtypeStruct((_N, _D), jnp.float32),
        jax.ShapeDtypeStruct((8, _D), jnp.float32),
    ],
)


def _bn_stats(st_ref):
    m = st_ref[0:1] / _N
    v = st_ref[1:2] / _N - m * m
    inv = lax.rsqrt(v + 1e-5)
    return m, inv


def _bn_body(z_ref, st_ref, g_ref, b_ref, o_ref):
    m, inv = _bn_stats(st_ref)
    o_ref[...] = jnp.maximum(
        (z_ref[...] - m) * inv * g_ref[...] + b_ref[...], 0.0)


_bn = pl.pallas_call(
    _bn_body,
    grid=(_R,),
    in_specs=[
        pl.BlockSpec((_BLK, _D), lambda i: (i, 0)),
        pl.BlockSpec((8, _D), lambda i: (0, 0)),
        pl.BlockSpec((1, _D), lambda i: (0, 0)),
        pl.BlockSpec((1, _D), lambda i: (0, 0)),
    ],
    out_specs=pl.BlockSpec((_BLK, _D), lambda i: (i, 0)),
    out_shape=jax.ShapeDtypeStruct((_N, _D), jnp.float32),
)


def _bnpool_body(z_ref, st_ref, g_ref, b_ref, bat_ref, o_ref, cnt_ref):
    i = pl.program_id(0)
    m, inv = _bn_stats(st_ref)
    zn = jnp.maximum((z_ref[...] - m) * inv * g_ref[...] + b_ref[...], 0.0)
    bvec = bat_ref[0, 0, :]
    gid = lax.broadcasted_iota(jnp.int32, (_BLK, _G), 1)
    mask = (bvec[:, None] == gid).astype(jnp.float32)
    pool = lax.dot_general(mask, zn, (((0,), (0,)), ((), ())),
                           preferred_element_type=jnp.float32)
    cnt = lax.dot_general(mask, jnp.ones((_BLK, _D), jnp.float32),
                          (((0,), (0,)), ((), ())),
                          preferred_element_type=jnp.float32)

    @pl.when(i == 0)
    def _():
        o_ref[...] = pool
        cnt_ref[...] = cnt

    @pl.when(i > 0)
    def _():
        o_ref[...] += pool
        cnt_ref[...] += cnt

    @pl.when(i == _R - 1)
    def _():
        o_ref[...] = o_ref[...] / jnp.maximum(cnt_ref[...], 1.0)


_bnpool = pl.pallas_call(
    _bnpool_body,
    grid=(_R,),
    in_specs=[
        pl.BlockSpec((_BLK, _D), lambda i: (i, 0)),
        pl.BlockSpec((8, _D), lambda i: (0, 0)),
        pl.BlockSpec((1, _D), lambda i: (0, 0)),
        pl.BlockSpec((1, _D), lambda i: (0, 0)),
        pl.BlockSpec((1, 1, _BLK), lambda i: (i, 0, 0)),
    ],
    out_specs=pl.BlockSpec((_G, _D), lambda i: (0, 0)),
    out_shape=jax.ShapeDtypeStruct((_G, _D), jnp.float32),
    scratch_shapes=[pltpu.VMEM((_G, _D), jnp.float32)],
)


def kernel(x, edge_index, batch,
           w1_1, b1_1, w1_2, b1_2,
           w2_1, b2_1, w2_2, b2_2,
           w3_1, b3_1, w3_2, b3_2,
           bn1_g, bn1_b, bn2_g, bn2_b, bn3_g, bn3_b):
    src = edge_index[0].reshape(2, _NC, _NS, _HCHUNK, _CHUNK)
    dst = edge_index[1].reshape(2, _NC, _NS, _HCHUNK, _CHUNK)
    zero = jnp.zeros((_NP, _D), jnp.float32)
    bat3 = batch.reshape(_R, 1, _BLK)

    layers = [
        (w1_1, b1_1, w1_2, b1_2),
        (w2_1, b2_1, w2_2, b2_2),
        (w3_1, b3_1, w3_2, b3_2),
    ]
    bns = [(bn1_g, bn1_b), (bn2_g, bn2_b)]
    h = x
    for li, (wa, ba, wb, bb) in enumerate(layers):
        agg = _get_sc_agg()(h, src, dst, zero)
        z, st = _mlp(agg, wa, ba.reshape(1, _D), wb, bb.reshape(1, _D))
        if li < 2:
            g, b = bns[li]
            h = _bn(z, st, g.reshape(1, _D), b.reshape(1, _D))
    return _bnpool(z, st, bn3_g.reshape(1, _D), bn3_b.reshape(1, _D), bat3)


# speedup vs baseline: 10.8909x; 1.0343x over previous
import functools

import jax
import jax.numpy as jnp
from jax import lax
from jax.experimental import pallas as pl
from jax.experimental.pallas import tpu as pltpu
from jax.experimental.pallas import tpu_sc as plsc

_N = 10000
_E = 320000
_D = 128
_G = 64

_NC = 2
_NS = 16
_EPW = _E // (_NC * _NS)
_CHUNK = 100
_NCHUNK = _EPW // _CHUNK
_HCHUNK = _NCHUNK // 2
_NP = 10240
_RPT = _NP // _NS
_HTAIL = _N - (_NS - 1) * _RPT

_BLK = 1000
_R = _N // _BLK


def _sc_agg_body(h_hbm, src_hbm, dst_hbm, zero_hbm, out_hbm,
                 src_v, dst_v, buf0, buf1, acc_sh, sem0, sem1):
    c = lax.axis_index("c")
    s = lax.axis_index("s")
    r0 = s * _RPT

    @pl.when(c == 0)
    def _():
        @pl.when(s < _NS - 1)
        def _():
            pltpu.sync_copy(h_hbm.at[pl.ds(r0, _RPT)],
                            acc_sh.at[pl.ds(r0, _RPT)])

        @pl.when(s == _NS - 1)
        def _():
            pltpu.sync_copy(h_hbm.at[pl.ds(_N - _HTAIL, _HTAIL)],
                            acc_sh.at[pl.ds(_N - _HTAIL, _HTAIL)])
            pltpu.sync_copy(zero_hbm.at[pl.ds(0, _NP - _N)],
                            acc_sh.at[pl.ds(_N, _NP - _N)])

    @pl.when(c == 1)
    def _():
        pltpu.sync_copy(zero_hbm.at[pl.ds(r0, _RPT)],
                        acc_sh.at[pl.ds(r0, _RPT)])

    plsc.subcore_barrier()

    for half in range(2):
        pltpu.sync_copy(src_hbm.at[half, c, s], src_v)
        pltpu.sync_copy(dst_hbm.at[half, c, s], dst_v)
        pltpu.async_copy(h_hbm.at[src_v.at[0]], buf0, sem0)
        pltpu.async_copy(h_hbm.at[src_v.at[1]], buf1, sem1)

        def body(p, carry):
            j = 2 * p
            pltpu.make_async_copy(h_hbm.at[src_v.at[j]], buf0, sem0).wait()
            pltpu.sync_copy(buf0, acc_sh.at[dst_v.at[j]], add=True)
            pltpu.async_copy(h_hbm.at[src_v.at[j + 2]], buf0, sem0)
            pltpu.make_async_copy(h_hbm.at[src_v.at[j + 1]], buf1,
                                  sem1).wait()
            pltpu.sync_copy(buf1, acc_sh.at[dst_v.at[j + 1]], add=True)
            pltpu.async_copy(h_hbm.at[src_v.at[j + 3]], buf1, sem1)
            return carry

        lax.fori_loop(0, _HCHUNK // 2 - 1, body, 0)
        j = _HCHUNK - 2
        pltpu.make_async_copy(h_hbm.at[src_v.at[j]], buf0, sem0).wait()
        pltpu.sync_copy(buf0, acc_sh.at[dst_v.at[j]], add=True)
        pltpu.make_async_copy(h_hbm.at[src_v.at[j + 1]], buf1, sem1).wait()
        pltpu.sync_copy(buf1, acc_sh.at[dst_v.at[j + 1]], add=True)

    plsc.subcore_barrier()
    pltpu.sync_copy(acc_sh.at[pl.ds(r0, _RPT)], out_hbm.at[c, pl.ds(r0, _RPT)])


@functools.lru_cache(maxsize=None)
def _get_sc_agg():
    return pl.kernel(
        _sc_agg_body,
        out_type=jax.ShapeDtypeStruct((_NC, _NP, _D), jnp.float32),
        mesh=plsc.VectorSubcoreMesh(
            core_axis_name="c", subcore_axis_name="s", num_cores=_NC),
        scratch_types=[
            pltpu.VMEM((_HCHUNK, _CHUNK), jnp.int32),
            pltpu.VMEM((_HCHUNK, _CHUNK), jnp.int32),
            pltpu.VMEM((_CHUNK, _D), jnp.float32),
            pltpu.VMEM((_CHUNK, _D), jnp.float32),
            pltpu.VMEM_SHARED((_NP, _D), jnp.float32),
            pltpu.SemaphoreType.DMA,
            pltpu.SemaphoreType.DMA,
        ],
    )


def _bn_from_stats(st):
    m = st[0:1] / _N
    v = st[1:2] / _N - m * m
    inv = lax.rsqrt(v + 1e-5)
    return m, inv


def _mlp_pass1(i, agg_ref, w1_ref, b1_ref, w2_ref, b2_ref, z_buf, st_buf):
    sm = agg_ref[0] + agg_ref[1]
    a = jnp.maximum(
        jnp.dot(sm, w1_ref[...], preferred_element_type=jnp.float32)
        + b1_ref[...], 0.0)
    z = (jnp.dot(a, w2_ref[...], preferred_element_type=jnp.float32)
         + b2_ref[...])
    r0 = pl.multiple_of(i * _BLK, _BLK)
    z_buf[pl.ds(r0, _BLK), :] = z
    part = jnp.concatenate(
        [jnp.sum(z, axis=0, keepdims=True),
         jnp.sum(z * z, axis=0, keepdims=True)], axis=0)

    @pl.when(i == 0)
    def _():
        st_buf[...] = part

    @pl.when(i > 0)
    def _():
        st_buf[...] += part


def _layer_body(agg_ref, w1_ref, b1_ref, w2_ref, b2_ref, g_ref, b_ref,
                o_ref, z_buf, st_buf):
    i = pl.program_id(0)

    @pl.when(i < _R)
    def _():
        _mlp_pass1(i, agg_ref, w1_ref, b1_ref, w2_ref, b2_ref, z_buf, st_buf)

    @pl.when(i >= _R)
    def _():
        j = i - _R
        r0 = pl.multiple_of(j * _BLK, _BLK)
        z = z_buf[pl.ds(r0, _BLK), :]
        m, inv = _bn_from_stats(st_buf[...])
        o_ref[...] = jnp.maximum(
            (z - m) * inv * g_ref[...] + b_ref[...], 0.0)


_layer = pl.pallas_call(
    _layer_body,
    grid=(2 * _R,),
    in_specs=[
        pl.BlockSpec((_NC, _BLK, _D),
                     lambda i: (0, jnp.where(i < _R, i, 0), 0)),
        pl.BlockSpec((_D, _D), lambda i: (0, 0)),
        pl.BlockSpec((1, _D), lambda i: (0, 0)),
        pl.BlockSpec((_D, _D), lambda i: (0, 0)),
        pl.BlockSpec((1, _D), lambda i: (0, 0)),
        pl.BlockSpec((1, _D), lambda i: (0, 0)),
        pl.BlockSpec((1, _D), lambda i: (0, 0)),
    ],
    out_specs=pl.BlockSpec((_BLK, _D),
                           lambda i: (jnp.where(i < _R, 0, i - _R), 0)),
    out_shape=jax.ShapeDtypeStruct((_N, _D), jnp.float32),
    scratch_shapes=[
        pltpu.VMEM((_N, _D), jnp.float32),
        pltpu.VMEM((2, _D), jnp.float32),
    ],
)


def _layer_pool_body(agg_ref, w1_ref, b1_ref, w2_ref, b2_ref, g_ref, b_ref,
                     bat_ref, o_ref, z_buf, st_buf, cnt_buf):
    i = pl.program_id(0)

    @pl.when(i < _R)
    def _():
        _mlp_pass1(i, agg_ref, w1_ref, b1_ref, w2_ref, b2_ref, z_buf, st_buf)

    @pl.when(i >= _R)
    def _():
        j = i - _R
        r0 = pl.multiple_of(j * _BLK, _BLK)
        z = z_buf[pl.ds(r0, _BLK), :]
        m, inv = _bn_from_stats(st_buf[...])
        zn = jnp.maximum((z - m) * inv * g_ref[...] + b_ref[...], 0.0)
        bvec = bat_ref[0, 0, :]
        gid = lax.broadcasted_iota(jnp.int32, (_BLK, _G), 1)
        mask = (bvec[:, None] == gid).astype(jnp.float32)
        pool = lax.dot_general(mask, zn, (((0,), (0,)), ((), ())),
                               preferred_element_type=jnp.float32)
        cnt = lax.dot_general(mask, jnp.ones((_BLK, _D), jnp.float32),
                              (((0,), (0,)), ((), ())),
                              preferred_element_type=jnp.float32)

        @pl.when(i == _R)
        def _():
            o_ref[...] = pool
            cnt_buf[...] = cnt

        @pl.when(i > _R)
        def _():
            o_ref[...] += pool
            cnt_buf[...] += cnt

        @pl.when(i == 2 * _R - 1)
        def _():
            o_ref[...] = o_ref[...] / jnp.maximum(cnt_buf[...], 1.0)


_layer_pool = pl.pallas_call(
    _layer_pool_body,
    grid=(2 * _R,),
    in_specs=[
        pl.BlockSpec((_NC, _BLK, _D),
                     lambda i: (0, jnp.where(i < _R, i, 0), 0)),
        pl.BlockSpec((_D, _D), lambda i: (0, 0)),
        pl.BlockSpec((1, _D), lambda i: (0, 0)),
        pl.BlockSpec((_D, _D), lambda i: (0, 0)),
        pl.BlockSpec((1, _D), lambda i: (0, 0)),
        pl.BlockSpec((1, _D), lambda i: (0, 0)),
        pl.BlockSpec((1, _D), lambda i: (0, 0)),
        pl.BlockSpec((1, 1, _BLK),
                     lambda i: (jnp.where(i < _R, 0, i - _R), 0, 0)),
    ],
    out_specs=pl.BlockSpec((_G, _D), lambda i: (0, 0)),
    out_shape=jax.ShapeDtypeStruct((_G, _D), jnp.float32),
    scratch_shapes=[
        pltpu.VMEM((_N, _D), jnp.float32),
        pltpu.VMEM((2, _D), jnp.float32),
        pltpu.VMEM((_G, _D), jnp.float32),
    ],
)


def kernel(x, edge_index, batch,
           w1_1, b1_1, w1_2, b1_2,
           w2_1, b2_1, w2_2, b2_2,
           w3_1, b3_1, w3_2, b3_2,
           bn1_g, bn1_b, bn2_g, bn2_b, bn3_g, bn3_b):
    src = edge_index[0].reshape(2, _NC, _NS, _HCHUNK, _CHUNK)
    dst = edge_index[1].reshape(2, _NC, _NS, _HCHUNK, _CHUNK)
    zero = jnp.zeros((_NP, _D), jnp.float32)
    bat3 = batch.reshape(_R, 1, _BLK)

    layers = [
        (w1_1, b1_1, w1_2, b1_2),
        (w2_1, b2_1, w2_2, b2_2),
        (w3_1, b3_1, w3_2, b3_2),
    ]
    bns = [(bn1_g, bn1_b), (bn2_g, bn2_b)]
    h = x
    for li, (wa, ba, wb, bb) in enumerate(layers):
        agg = _get_sc_agg()(h, src, dst, zero)
        if li < 2:
            g, b = bns[li]
            h = _layer(agg, wa, ba.reshape(1, _D), wb, bb.reshape(1, _D),
                       g.reshape(1, _D), b.reshape(1, _D))
        else:
            out = _layer_pool(agg, wa, ba.reshape(1, _D), wb,
                              bb.reshape(1, _D), bn3_g.reshape(1, _D),
                              bn3_b.reshape(1, _D), bat3)
    return out
